# baseline (speedup 1.0000x reference)
import functools

import jax
import jax.numpy as jnp
from jax import lax
from jax.experimental import pallas as pl
from jax.experimental.pallas import tpu as pltpu
from jax.experimental.pallas import tpu_sc as plsc

DIM = 128
LANES = 16
NC = 2
NS = 16
NW = NC * NS
CHUNK = 64
NSLOT = 2
ROWU = 4

_GATHER_DNUMS = lax.GatherDimensionNumbers(
    offset_dims=(), collapsed_slice_dims=(0,), start_index_map=(0,))


def _shuffle(v, idx):
    return lax.gather(
        v, idx[:, None], dimension_numbers=_GATHER_DNUMS, slice_sizes=(1,),
        mode=lax.GatherScatterMode.PROMISE_IN_BOUNDS)


def _transe_sc(idx_all, ent, rel):
    B = idx_all.shape[0] // 3
    per_w = B // NW
    n_chunks = per_w // CHUNK
    blk = 3 * per_w
    roff = 2 * per_w

    mesh = plsc.VectorSubcoreMesh(core_axis_name="c", subcore_axis_name="s")

    @functools.partial(
        pl.kernel,
        mesh=mesh,
        out_type=jax.ShapeDtypeStruct((B,), jnp.float32),
        scratch_types=[
            pltpu.VMEM((blk,), jnp.int32),
            pltpu.VMEM((NSLOT, 2 * CHUNK, DIM), jnp.float32),
            pltpu.VMEM((NSLOT, CHUNK, DIM), jnp.float32),
            pltpu.VMEM((per_w,), jnp.float32),
            pltpu.SemaphoreType.DMA,
            pltpu.SemaphoreType.DMA,
        ],
    )
    def k(idx_hbm, ent_hbm, rel_hbm, out_hbm,
          idxv, htbuf, rbuf, outv, sem0, sem1):
        sems = (sem0, sem1)
        lane = lax.iota(jnp.int32, LANES)
        wid = lax.axis_index("s") * NC + lax.axis_index("c")
        pltpu.sync_copy(idx_hbm.at[pl.ds(wid * blk, blk)], idxv)

        def fire(c, slot):
            pltpu.async_copy(
                ent_hbm.at[idxv.at[pl.ds(2 * c * CHUNK, 2 * CHUNK)]],
                htbuf.at[slot], sems[slot])
            pltpu.async_copy(
                rel_hbm.at[idxv.at[pl.ds(roff + c * CHUNK, CHUNK)]],
                rbuf.at[slot], sems[slot])

        def drain(c, slot):
            pltpu.make_async_copy(
                ent_hbm.at[idxv.at[pl.ds(2 * c * CHUNK, 2 * CHUNK)]],
                htbuf.at[slot], sems[slot]).wait()
            pltpu.make_async_copy(
                rel_hbm.at[idxv.at[pl.ds(roff + c * CHUNK, CHUNK)]],
                rbuf.at[slot], sems[slot]).wait()

        perms = [lane ^ sh for sh in (8, 4, 2, 1)]

        def compute(c, slot):
            def rows_body(g, res):
                sub = (g % (LANES // ROWU)) * ROWU
                for kk in range(ROWU):
                    i = g * ROWU + kk
                    acc = jnp.zeros((LANES,), jnp.float32)
                    for j in range(DIM // LANES):
                        sl = pl.ds(j * LANES, LANES)
                        h = htbuf[slot, i, sl]
                        t = htbuf[slot, CHUNK + i, sl]
                        r = rbuf[slot, i, sl]
                        acc = acc + jnp.abs(h + r - t)
                    for p in perms:
                        acc = acc + _shuffle(acc, p)
                    res = jnp.where(lane == sub + kk, -acc, res)

                @pl.when(sub == LANES - ROWU)
                def _():
                    outv[pl.ds(c * CHUNK + (g // (LANES // ROWU)) * LANES,
                               LANES)] = res

                return res

            lax.fori_loop(0, CHUNK // ROWU, rows_body,
                          jnp.zeros((LANES,), jnp.float32))

        fire(0, 0)

        def chunk_body(c, _):
            par = c % 2

            @pl.when(par == 0)
            def _():
                drain(c, 0)

                @pl.when(c + 1 < n_chunks)
                def _():
                    fire(c + 1, 1)

            @pl.when(par == 1)
            def _():
                drain(c, 1)

                @pl.when(c + 1 < n_chunks)
                def _():
                    fire(c + 1, 0)

            compute(c, par)
            return 0

        lax.fori_loop(0, n_chunks, chunk_body, 0)

        pltpu.sync_copy(outv, out_hbm.at[pl.ds(wid * per_w, per_w)])

    return k(idx_all, ent, rel)


def kernel(rel_ids, head, tail, ent, rel):
    per_w = head.shape[0] // NW
    n_chunks = per_w // CHUNK
    ht = jnp.stack(
        [head.astype(jnp.int32).reshape(NW, n_chunks, CHUNK),
         tail.astype(jnp.int32).reshape(NW, n_chunks, CHUNK)],
        axis=2,
    ).reshape(NW, 2 * per_w)
    idx_all = jnp.concatenate(
        [ht, rel_ids.astype(jnp.int32).reshape(NW, per_w)], axis=1
    ).reshape(-1)
    return _transe_sc(idx_all, ent, rel)


# speedup vs baseline: 1.0034x; 1.0034x over previous
import functools

import jax
import jax.numpy as jnp
from jax import lax
from jax.experimental import pallas as pl
from jax.experimental.pallas import tpu as pltpu
from jax.experimental.pallas import tpu_sc as plsc

DIM = 128
LANES = 16
NC = 2
NS = 16
NW = NC * NS
CHUNK = 64
NSLOT = 2
ROWU = 4

_GATHER_DNUMS = lax.GatherDimensionNumbers(
    offset_dims=(), collapsed_slice_dims=(0,), start_index_map=(0,))


def _shuffle(v, idx):
    return lax.gather(
        v, idx[:, None], dimension_numbers=_GATHER_DNUMS, slice_sizes=(1,),
        mode=lax.GatherScatterMode.PROMISE_IN_BOUNDS)


def _transe_sc(idx_all, ent, rel):
    B = idx_all.shape[0] // 3
    per_w = B // NW
    n_chunks = per_w // CHUNK
    blk = 3 * per_w
    roff = 2 * per_w

    mesh = plsc.VectorSubcoreMesh(core_axis_name="c", subcore_axis_name="s")

    @functools.partial(
        pl.kernel,
        mesh=mesh,
        out_type=jax.ShapeDtypeStruct((B,), jnp.float32),
        scratch_types=[
            pltpu.VMEM((blk,), jnp.int32),
            pltpu.VMEM((NSLOT, 2 * CHUNK, DIM), jnp.float32),
            pltpu.VMEM((NSLOT, CHUNK, DIM), jnp.float32),
            pltpu.VMEM((per_w,), jnp.float32),
            pltpu.SemaphoreType.DMA,
            pltpu.SemaphoreType.DMA,
        ],
    )
    def k(idx_hbm, ent_hbm, rel_hbm, out_hbm,
          idxv, htbuf, rbuf, outv, sem0, sem1):
        sems = (sem0, sem1)
        lane = lax.iota(jnp.int32, LANES)
        wid = lax.axis_index("s") * NC + lax.axis_index("c")
        pltpu.sync_copy(idx_hbm.at[pl.ds(wid * blk, 2 * CHUNK)],
                        idxv.at[pl.ds(0, 2 * CHUNK)])
        pltpu.async_copy(
            ent_hbm.at[idxv.at[pl.ds(0, 2 * CHUNK)]], htbuf.at[0], sem0)
        pltpu.sync_copy(idx_hbm.at[pl.ds(wid * blk + 2 * CHUNK,
                                         blk - 2 * CHUNK)],
                        idxv.at[pl.ds(2 * CHUNK, blk - 2 * CHUNK)])

        def fire(c, slot):
            pltpu.async_copy(
                ent_hbm.at[idxv.at[pl.ds(2 * c * CHUNK, 2 * CHUNK)]],
                htbuf.at[slot], sems[slot])
            pltpu.async_copy(
                rel_hbm.at[idxv.at[pl.ds(roff + c * CHUNK, CHUNK)]],
                rbuf.at[slot], sems[slot])

        def drain(c, slot):
            pltpu.make_async_copy(
                ent_hbm.at[idxv.at[pl.ds(2 * c * CHUNK, 2 * CHUNK)]],
                htbuf.at[slot], sems[slot]).wait()
            pltpu.make_async_copy(
                rel_hbm.at[idxv.at[pl.ds(roff + c * CHUNK, CHUNK)]],
                rbuf.at[slot], sems[slot]).wait()

        perms = [lane ^ sh for sh in (8, 4, 2, 1)]

        def compute(c, slot):
            def rows_body(g, res):
                sub = (g % (LANES // ROWU)) * ROWU
                for kk in range(ROWU):
                    i = g * ROWU + kk
                    acc = jnp.zeros((LANES,), jnp.float32)
                    for j in range(DIM // LANES):
                        sl = pl.ds(j * LANES, LANES)
                        h = htbuf[slot, i, sl]
                        t = htbuf[slot, CHUNK + i, sl]
                        r = rbuf[slot, i, sl]
                        acc = acc + jnp.abs(h + r - t)
                    for p in perms:
                        acc = acc + _shuffle(acc, p)
                    res = jnp.where(lane == sub + kk, -acc, res)

                @pl.when(sub == LANES - ROWU)
                def _():
                    outv[pl.ds(c * CHUNK + (g // (LANES // ROWU)) * LANES,
                               LANES)] = res

                return res

            lax.fori_loop(0, CHUNK // ROWU, rows_body,
                          jnp.zeros((LANES,), jnp.float32))

        pltpu.async_copy(
            rel_hbm.at[idxv.at[pl.ds(roff, CHUNK)]], rbuf.at[0], sem0)

        def chunk_body(c, _):
            par = c % 2

            @pl.when(par == 0)
            def _():
                drain(c, 0)

                @pl.when(c + 1 < n_chunks)
                def _():
                    fire(c + 1, 1)

            @pl.when(par == 1)
            def _():
                drain(c, 1)

                @pl.when(c + 1 < n_chunks)
                def _():
                    fire(c + 1, 0)

            compute(c, par)
            return 0

        lax.fori_loop(0, n_chunks, chunk_body, 0)

        pltpu.sync_copy(outv, out_hbm.at[pl.ds(wid * per_w, per_w)])

    return k(idx_all, ent, rel)


def kernel(rel_ids, head, tail, ent, rel):
    per_w = head.shape[0] // NW
    n_chunks = per_w // CHUNK
    ht = jnp.stack(
        [head.astype(jnp.int32).reshape(NW, n_chunks, CHUNK),
         tail.astype(jnp.int32).reshape(NW, n_chunks, CHUNK)],
        axis=2,
    ).reshape(NW, 2 * per_w)
    idx_all = jnp.concatenate(
        [ht, rel_ids.astype(jnp.int32).reshape(NW, per_w)], axis=1
    ).reshape(-1)
    return _transe_sc(idx_all, ent, rel)
